# Initial kernel scaffold; baseline (speedup 1.0000x reference)
import jax
import jax.numpy as jnp
from jax.experimental import pallas as pl


def kernel(x, edge_index, W1_0, b1_0, W2_0, b2_0, gamma_0, beta_0, mean_0, var_0, W1_1, b1_1, W2_1, b2_1, gamma_1, beta_1, mean_1, var_1, W1_2, b1_2, W2_2, b2_2, gamma_2, beta_2, mean_2, var_2, Wc, bc):
    raise NotImplementedError("write your pallas kernel here")


# speedup vs baseline: 5.6363x; 5.6363x over previous
import functools

import jax
import jax.numpy as jnp
from jax import lax
from jax.experimental import pallas as pl
from jax.experimental.pallas import tpu as pltpu
from jax.experimental.pallas import tpu_sc as plsc

N = 10000
E = 320000
D = 128
D_OUT = 64

NC = 2
NS = 16
NW = NC * NS
EPW = E // NW
CHUNK = 128
NFULL = EPW // CHUNK
TAIL = EPW - NFULL * CHUNK
RPW = 624
RTAIL = N - NS * RPW


def _agg_sc(h, src, dst, zeros):
    mesh = plsc.VectorSubcoreMesh(core_axis_name="c", subcore_axis_name="s")

    @functools.partial(
        pl.kernel,
        out_type=jax.ShapeDtypeStruct((NC, N, D), jnp.float32),
        mesh=mesh,
        scratch_types=[
            pltpu.VMEM((CHUNK,), jnp.int32),
            pltpu.VMEM((CHUNK,), jnp.int32),
            pltpu.VMEM((CHUNK, D), jnp.float32),
            pltpu.VMEM((TAIL,), jnp.int32),
            pltpu.VMEM((TAIL,), jnp.int32),
            pltpu.VMEM((TAIL, D), jnp.float32),
            pltpu.VMEM_SHARED((N, D), jnp.float32),
            pltpu.SemaphoreType.DMA,
        ],
    )
    def body(h_hbm, src_hbm, dst_hbm, z_hbm, out_hbm,
             src_v, dst_v, rows_v, srct_v, dstt_v, rowst_v, acc_sh, sem):
        c = lax.axis_index("c")
        s = lax.axis_index("s")
        wid = c * NS + s

        pltpu.sync_copy(z_hbm.at[pl.ds(s * RPW, RPW)],
                        acc_sh.at[pl.ds(s * RPW, RPW)])

        @pl.when(s == 0)
        def _():
            pltpu.sync_copy(z_hbm.at[pl.ds(NS * RPW, RTAIL)],
                            acc_sh.at[pl.ds(NS * RPW, RTAIL)])

        plsc.subcore_barrier()

        base = wid * EPW

        def step(i, carry):
            off = pl.multiple_of(base + i * CHUNK, 8)
            pltpu.sync_copy(src_hbm.at[pl.ds(off, CHUNK)], src_v)
            pltpu.sync_copy(dst_hbm.at[pl.ds(off, CHUNK)], dst_v)
            pltpu.async_copy(h_hbm.at[src_v], rows_v, sem).wait()
            pltpu.sync_copy(rows_v, acc_sh.at[dst_v], add=True)
            return carry

        lax.fori_loop(0, NFULL, step, 0)

        offt = pl.multiple_of(base + NFULL * CHUNK, 8)
        pltpu.sync_copy(src_hbm.at[pl.ds(offt, TAIL)], srct_v)
        pltpu.sync_copy(dst_hbm.at[pl.ds(offt, TAIL)], dstt_v)
        pltpu.async_copy(h_hbm.at[srct_v], rowst_v, sem).wait()
        pltpu.sync_copy(rowst_v, acc_sh.at[dstt_v], add=True)

        plsc.subcore_barrier()
        pltpu.sync_copy(acc_sh.at[pl.ds(s * RPW, RPW)],
                        out_hbm.at[c].at[pl.ds(s * RPW, RPW)])

        @pl.when(s == 0)
        def _():
            pltpu.sync_copy(acc_sh.at[pl.ds(NS * RPW, RTAIL)],
                            out_hbm.at[c].at[pl.ds(NS * RPW, RTAIL)])

    return body(h, src, dst, zeros)


BLK = 1000


def _mlp_body(x_ref, a_ref, w1_ref, b1_ref, w2_ref, b2_ref, sc_ref, sh_ref,
              out_ref):
    h = x_ref[...] + a_ref[0] + a_ref[1]
    t = jnp.dot(h, w1_ref[...], preferred_element_type=jnp.float32) + b1_ref[...]
    t = jnp.maximum(t, 0.0)
    t = jnp.dot(t, w2_ref[...], preferred_element_type=jnp.float32) + b2_ref[...]
    out_ref[...] = jnp.maximum(t * sc_ref[...] + sh_ref[...], 0.0)


def _mlp_tc(x, agg, w1, b1, w2, b2, scale, shift):
    grid = N // BLK
    full = lambda *_: (0, 0)
    return pl.pallas_call(
        _mlp_body,
        grid=(grid,),
        in_specs=[
            pl.BlockSpec((BLK, D), lambda i: (i, 0)),
            pl.BlockSpec((NC, BLK, D), lambda i: (0, i, 0)),
            pl.BlockSpec((D, D), full),
            pl.BlockSpec((1, D), full),
            pl.BlockSpec((D, D), full),
            pl.BlockSpec((1, D), full),
            pl.BlockSpec((1, D), full),
            pl.BlockSpec((1, D), full),
        ],
        out_specs=pl.BlockSpec((BLK, D), lambda i: (i, 0)),
        out_shape=jax.ShapeDtypeStruct((N, D), jnp.float32),
    )(x, agg, w1, b1, w2, b2, scale, shift)


def _mlp3_body(x_ref, a_ref, w1_ref, b1_ref, w2_ref, b2_ref, sc_ref, sh_ref,
               wc_ref, bc_ref, out_ref, acc_ref):
    i = pl.program_id(0)
    h = x_ref[...] + a_ref[0] + a_ref[1]
    t = jnp.dot(h, w1_ref[...], preferred_element_type=jnp.float32) + b1_ref[...]
    t = jnp.maximum(t, 0.0)
    t = jnp.dot(t, w2_ref[...], preferred_element_type=jnp.float32) + b2_ref[...]
    t = jnp.maximum(t * sc_ref[...] + sh_ref[...], 0.0)
    psum = jnp.sum(t, axis=0, keepdims=True)

    @pl.when(i == 0)
    def _():
        acc_ref[...] = psum

    @pl.when(i > 0)
    def _():
        acc_ref[...] += psum

    @pl.when(i == pl.num_programs(0) - 1)
    def _():
        out_ref[...] = (jnp.dot(acc_ref[...], wc_ref[...],
                                preferred_element_type=jnp.float32)
                        + bc_ref[...])


def _mlp3_tc(x, agg, w1, b1, w2, b2, scale, shift, wc, bc):
    grid = N // BLK
    full = lambda *_: (0, 0)
    return pl.pallas_call(
        _mlp3_body,
        grid=(grid,),
        in_specs=[
            pl.BlockSpec((BLK, D), lambda i: (i, 0)),
            pl.BlockSpec((NC, BLK, D), lambda i: (0, i, 0)),
            pl.BlockSpec((D, D), full),
            pl.BlockSpec((1, D), full),
            pl.BlockSpec((D, D), full),
            pl.BlockSpec((1, D), full),
            pl.BlockSpec((1, D), full),
            pl.BlockSpec((1, D), full),
            pl.BlockSpec((D, D_OUT), full),
            pl.BlockSpec((1, D_OUT), full),
        ],
        out_specs=pl.BlockSpec((1, D_OUT), full),
        out_shape=jax.ShapeDtypeStruct((1, D_OUT), jnp.float32),
        scratch_shapes=[pltpu.VMEM((1, D), jnp.float32)],
    )(x, agg, w1, b1, w2, b2, scale, shift, wc, bc)


def kernel(x, edge_index,
           W1_0, b1_0, W2_0, b2_0, gamma_0, beta_0, mean_0, var_0,
           W1_1, b1_1, W2_1, b2_1, gamma_1, beta_1, mean_1, var_1,
           W1_2, b1_2, W2_2, b2_2, gamma_2, beta_2, mean_2, var_2,
           Wc, bc):
    src = edge_index[0]
    dst = edge_index[1]
    zeros = jnp.zeros((N, D), jnp.float32)

    def fold_bn(gamma, beta, mean, var):
        scale = gamma / jnp.sqrt(var + 1e-5)
        shift = beta - mean * scale
        return scale.reshape(1, D), shift.reshape(1, D)

    sc0, sh0 = fold_bn(gamma_0, beta_0, mean_0, var_0)
    sc1, sh1 = fold_bn(gamma_1, beta_1, mean_1, var_1)
    sc2, sh2 = fold_bn(gamma_2, beta_2, mean_2, var_2)

    h = x
    agg = _agg_sc(h, src, dst, zeros)
    h = _mlp_tc(h, agg, W1_0, b1_0.reshape(1, D), W2_0, b2_0.reshape(1, D), sc0, sh0)
    agg = _agg_sc(h, src, dst, zeros)
    h = _mlp_tc(h, agg, W1_1, b1_1.reshape(1, D), W2_1, b2_1.reshape(1, D), sc1, sh1)
    agg = _agg_sc(h, src, dst, zeros)
    return _mlp3_tc(h, agg, W1_2, b1_2.reshape(1, D), W2_2, b2_2.reshape(1, D),
                    sc2, sh2, Wc, bc.reshape(1, D_OUT))


# speedup vs baseline: 11.7105x; 2.0777x over previous
import functools

import jax
import jax.numpy as jnp
from jax import lax
from jax.experimental import pallas as pl
from jax.experimental.pallas import tpu as pltpu
from jax.experimental.pallas import tpu_sc as plsc

N = 10000
E = 320000
D = 128
D_OUT = 64

NC = 2
NS = 16
NW = NC * NS
CHUNK = 128
NCHUNK = E // CHUNK
CPW = NCHUNK // NW
NEXTRA = NCHUNK - CPW * NW
EXBASE = CPW * NW
RPW = 624
RTAIL = N - NS * RPW
NSLOT = 3


def _agg_sc(h, src, dst, zeros):
    mesh = plsc.VectorSubcoreMesh(core_axis_name="c", subcore_axis_name="s")

    @functools.partial(
        pl.kernel,
        out_type=jax.ShapeDtypeStruct((NC, N, D), jnp.float32),
        mesh=mesh,
        scratch_types=(
            [pltpu.VMEM((CHUNK,), jnp.int32)] * NSLOT
            + [pltpu.VMEM((CHUNK,), jnp.int32)] * NSLOT
            + [pltpu.VMEM((CHUNK, D), jnp.float32)] * NSLOT
            + [pltpu.SemaphoreType.DMA] * (3 * NSLOT)
            + [pltpu.VMEM_SHARED((N, D), jnp.float32)]
        ),
    )
    def body(h_hbm, src_hbm, dst_hbm, z_hbm, out_hbm, *sc_refs):
        sv = sc_refs[0:3]
        dv = sc_refs[3:6]
        rows = sc_refs[6:9]
        sis = sc_refs[9:12]
        sid = sc_refs[12:15]
        sg = sc_refs[15:18]
        acc_sh = sc_refs[18]

        c_ax = lax.axis_index("c")
        s_ax = lax.axis_index("s")
        wid = c_ax * NS + s_ax
        base = wid * CPW

        def fire_idx(ch, k):
            off = pl.multiple_of(ch * CHUNK, 8)
            pltpu.async_copy(src_hbm.at[pl.ds(off, CHUNK)], sv[k], sis[k])
            pltpu.async_copy(dst_hbm.at[pl.ds(off, CHUNK)], dv[k], sid[k])

        def wait_is(k):
            pltpu.make_async_copy(src_hbm.at[pl.ds(0, CHUNK)], sv[k], sis[k]).wait()

        def wait_id(k):
            pltpu.make_async_copy(dst_hbm.at[pl.ds(0, CHUNK)], dv[k], sid[k]).wait()

        def fire_gather(k):
            pltpu.async_copy(h_hbm.at[sv[k]], rows[k], sg[k])

        def wait_gather(k):
            pltpu.make_async_copy(h_hbm.at[sv[k]], rows[k], sg[k]).wait()

        fire_idx(base, 0)
        fire_idx(base + 1, 1)
        wait_is(0)
        fire_gather(0)

        pltpu.sync_copy(z_hbm.at[pl.ds(s_ax * RPW, RPW)],
                        acc_sh.at[pl.ds(s_ax * RPW, RPW)])

        @pl.when(s_ax == 0)
        def _():
            pltpu.sync_copy(z_hbm.at[pl.ds(NS * RPW, RTAIL)],
                            acc_sh.at[pl.ds(NS * RPW, RTAIL)])

        plsc.subcore_barrier()

        def outer(o, carry):
            for k in range(NSLOT):
                c = o * NSLOT + k
                k1 = (k + 1) % NSLOT
                k2 = (k + 2) % NSLOT

                @pl.when(c + 2 < CPW)
                def _():
                    fire_idx(base + c + 2, k2)

                @pl.when(c + 1 < CPW)
                def _():
                    wait_is(k1)
                    fire_gather(k1)

                wait_gather(k)
                wait_id(k)
                pltpu.sync_copy(rows[k], acc_sh.at[dv[k]], add=True)
            return carry

        lax.fori_loop(0, CPW // NSLOT, outer, 0)

        @pl.when(wid < NEXTRA)
        def _():
            fire_idx(EXBASE + wid, 0)
            wait_is(0)
            fire_gather(0)
            wait_gather(0)
            wait_id(0)
            pltpu.sync_copy(rows[0], acc_sh.at[dv[0]], add=True)

        plsc.subcore_barrier()
        pltpu.sync_copy(acc_sh.at[pl.ds(s_ax * RPW, RPW)],
                        out_hbm.at[c_ax].at[pl.ds(s_ax * RPW, RPW)])

        @pl.when(s_ax == 0)
        def _():
            pltpu.sync_copy(acc_sh.at[pl.ds(NS * RPW, RTAIL)],
                            out_hbm.at[c_ax].at[pl.ds(NS * RPW, RTAIL)])

    return body(h, src, dst, zeros)


BLK = 1000


def _mlp_body(x_ref, a_ref, w1_ref, b1_ref, w2_ref, b2_ref, sc_ref, sh_ref,
              out_ref):
    h = x_ref[...] + a_ref[0] + a_ref[1]
    t = jnp.dot(h, w1_ref[...], preferred_element_type=jnp.float32) + b1_ref[...]
    t = jnp.maximum(t, 0.0)
    t = jnp.dot(t, w2_ref[...], preferred_element_type=jnp.float32) + b2_ref[...]
    out_ref[...] = jnp.maximum(t * sc_ref[...] + sh_ref[...], 0.0)


def _mlp_tc(x, agg, w1, b1, w2, b2, scale, shift):
    grid = N // BLK
    full = lambda *_: (0, 0)
    return pl.pallas_call(
        _mlp_body,
        grid=(grid,),
        in_specs=[
            pl.BlockSpec((BLK, D), lambda i: (i, 0)),
            pl.BlockSpec((NC, BLK, D), lambda i: (0, i, 0)),
            pl.BlockSpec((D, D), full),
            pl.BlockSpec((1, D), full),
            pl.BlockSpec((D, D), full),
            pl.BlockSpec((1, D), full),
            pl.BlockSpec((1, D), full),
            pl.BlockSpec((1, D), full),
        ],
        out_specs=pl.BlockSpec((BLK, D), lambda i: (i, 0)),
        out_shape=jax.ShapeDtypeStruct((N, D), jnp.float32),
    )(x, agg, w1, b1, w2, b2, scale, shift)


def _mlp3_body(x_ref, a_ref, w1_ref, b1_ref, w2_ref, b2_ref, sc_ref, sh_ref,
               wc_ref, bc_ref, out_ref, acc_ref):
    i = pl.program_id(0)
    h = x_ref[...] + a_ref[0] + a_ref[1]
    t = jnp.dot(h, w1_ref[...], preferred_element_type=jnp.float32) + b1_ref[...]
    t = jnp.maximum(t, 0.0)
    t = jnp.dot(t, w2_ref[...], preferred_element_type=jnp.float32) + b2_ref[...]
    t = jnp.maximum(t * sc_ref[...] + sh_ref[...], 0.0)
    psum = jnp.sum(t, axis=0, keepdims=True)

    @pl.when(i == 0)
    def _():
        acc_ref[...] = psum

    @pl.when(i > 0)
    def _():
        acc_ref[...] += psum

    @pl.when(i == pl.num_programs(0) - 1)
    def _():
        out_ref[...] = (jnp.dot(acc_ref[...], wc_ref[...],
                                preferred_element_type=jnp.float32)
                        + bc_ref[...])


def _mlp3_tc(x, agg, w1, b1, w2, b2, scale, shift, wc, bc):
    grid = N // BLK
    full = lambda *_: (0, 0)
    return pl.pallas_call(
        _mlp3_body,
        grid=(grid,),
        in_specs=[
            pl.BlockSpec((BLK, D), lambda i: (i, 0)),
            pl.BlockSpec((NC, BLK, D), lambda i: (0, i, 0)),
            pl.BlockSpec((D, D), full),
            pl.BlockSpec((1, D), full),
            pl.BlockSpec((D, D), full),
            pl.BlockSpec((1, D), full),
            pl.BlockSpec((1, D), full),
            pl.BlockSpec((1, D), full),
            pl.BlockSpec((D, D_OUT), full),
            pl.BlockSpec((1, D_OUT), full),
        ],
        out_specs=pl.BlockSpec((1, D_OUT), full),
        out_shape=jax.ShapeDtypeStruct((1, D_OUT), jnp.float32),
        scratch_shapes=[pltpu.VMEM((1, D), jnp.float32)],
    )(x, agg, w1, b1, w2, b2, scale, shift, wc, bc)


def kernel(x, edge_index,
           W1_0, b1_0, W2_0, b2_0, gamma_0, beta_0, mean_0, var_0,
           W1_1, b1_1, W2_1, b2_1, gamma_1, beta_1, mean_1, var_1,
           W1_2, b1_2, W2_2, b2_2, gamma_2, beta_2, mean_2, var_2,
           Wc, bc):
    src = edge_index[0]
    dst = edge_index[1]
    zeros = jnp.zeros((N, D), jnp.float32)

    def fold_bn(gamma, beta, mean, var):
        scale = gamma / jnp.sqrt(var + 1e-5)
        shift = beta - mean * scale
        return scale.reshape(1, D), shift.reshape(1, D)

    sc0, sh0 = fold_bn(gamma_0, beta_0, mean_0, var_0)
    sc1, sh1 = fold_bn(gamma_1, beta_1, mean_1, var_1)
    sc2, sh2 = fold_bn(gamma_2, beta_2, mean_2, var_2)

    h = x
    agg = _agg_sc(h, src, dst, zeros)
    h = _mlp_tc(h, agg, W1_0, b1_0.reshape(1, D), W2_0, b2_0.reshape(1, D), sc0, sh0)
    agg = _agg_sc(h, src, dst, zeros)
    h = _mlp_tc(h, agg, W1_1, b1_1.reshape(1, D), W2_1, b2_1.reshape(1, D), sc1, sh1)
    agg = _agg_sc(h, src, dst, zeros)
    return _mlp3_tc(h, agg, W1_2, b1_2.reshape(1, D), W2_2, b2_2.reshape(1, D),
                    sc2, sh2, Wc, bc.reshape(1, D_OUT))


# speedup vs baseline: 11.7895x; 1.0068x over previous
import functools

import jax
import jax.numpy as jnp
from jax import lax
from jax.experimental import pallas as pl
from jax.experimental.pallas import tpu as pltpu
from jax.experimental.pallas import tpu_sc as plsc

N = 10000
E = 320000
D = 128
D_OUT = 64

NC = 2
NS = 16
NW = NC * NS
CHUNK = 128
NCHUNK = E // CHUNK
CPW = NCHUNK // NW
NEXTRA = NCHUNK - CPW * NW
EXBASE = CPW * NW
RPW = 624
RTAIL = N - NS * RPW
NSLOT = 3


def _agg_sc(h, src, dst, zeros):
    mesh = plsc.VectorSubcoreMesh(core_axis_name="c", subcore_axis_name="s")

    @functools.partial(
        pl.kernel,
        out_type=jax.ShapeDtypeStruct((NC, N, D), jnp.float32),
        mesh=mesh,
        scratch_types=(
            [pltpu.VMEM((CHUNK,), jnp.int32)] * NSLOT
            + [pltpu.VMEM((CHUNK,), jnp.int32)] * NSLOT
            + [pltpu.VMEM((CHUNK, D), jnp.float32)] * NSLOT
            + [pltpu.SemaphoreType.DMA] * (4 * NSLOT)
            + [pltpu.VMEM_SHARED((N, D), jnp.float32)]
        ),
    )
    def body(h_hbm, src_hbm, dst_hbm, z_hbm, out_hbm, *sc_refs):
        sv = sc_refs[0:3]
        dv = sc_refs[3:6]
        rows = sc_refs[6:9]
        sis = sc_refs[9:12]
        sid = sc_refs[12:15]
        sg = sc_refs[15:18]
        ss = sc_refs[18:21]
        acc_sh = sc_refs[21]

        c_ax = lax.axis_index("c")
        s_ax = lax.axis_index("s")
        wid = c_ax * NS + s_ax
        base = wid * CPW

        def fire_idx(ch, k):
            off = pl.multiple_of(ch * CHUNK, 8)
            pltpu.async_copy(src_hbm.at[pl.ds(off, CHUNK)], sv[k], sis[k])
            pltpu.async_copy(dst_hbm.at[pl.ds(off, CHUNK)], dv[k], sid[k])

        def wait_is(k):
            pltpu.make_async_copy(src_hbm.at[pl.ds(0, CHUNK)], sv[k], sis[k]).wait()

        def wait_id(k):
            pltpu.make_async_copy(dst_hbm.at[pl.ds(0, CHUNK)], dv[k], sid[k]).wait()

        def fire_gather(k):
            pltpu.async_copy(h_hbm.at[sv[k]], rows[k], sg[k])

        def wait_gather(k):
            pltpu.make_async_copy(h_hbm.at[sv[k]], rows[k], sg[k]).wait()

        def fire_scatter(k):
            pltpu.async_copy(rows[k], acc_sh.at[dv[k]], ss[k], add=True)

        def wait_scatter(k):
            pltpu.make_async_copy(rows[k], acc_sh.at[dv[k]], ss[k]).wait()

        fire_idx(base, 0)
        fire_idx(base + 1, 1)
        wait_is(0)
        fire_gather(0)

        @pl.when(c_ax == 0)
        def _():
            pltpu.sync_copy(h_hbm.at[pl.ds(s_ax * RPW, RPW)],
                            acc_sh.at[pl.ds(s_ax * RPW, RPW)])

            @pl.when(s_ax == 0)
            def _():
                pltpu.sync_copy(h_hbm.at[pl.ds(NS * RPW, RTAIL)],
                                acc_sh.at[pl.ds(NS * RPW, RTAIL)])

        @pl.when(c_ax == 1)
        def _():
            pltpu.sync_copy(z_hbm.at[pl.ds(s_ax * RPW, RPW)],
                            acc_sh.at[pl.ds(s_ax * RPW, RPW)])

            @pl.when(s_ax == 0)
            def _():
                pltpu.sync_copy(z_hbm.at[pl.ds(NS * RPW, RTAIL)],
                                acc_sh.at[pl.ds(NS * RPW, RTAIL)])

        plsc.subcore_barrier()

        def outer(o, carry):
            for k in range(NSLOT):
                c = o * NSLOT + k
                k1 = (k + 1) % NSLOT
                k2 = (k + 2) % NSLOT

                @pl.when((c >= 1) & (c + 2 < CPW))
                def _():
                    wait_scatter(k2)

                @pl.when(c + 2 < CPW)
                def _():
                    fire_idx(base + c + 2, k2)

                @pl.when(c + 1 < CPW)
                def _():
                    wait_is(k1)
                    fire_gather(k1)

                wait_gather(k)
                wait_id(k)
                fire_scatter(k)
            return carry

        lax.fori_loop(0, CPW // NSLOT, outer, 0)

        for k in range(NSLOT):
            wait_scatter(k)

        @pl.when(wid < NEXTRA)
        def _():
            fire_idx(EXBASE + wid, 0)
            wait_is(0)
            fire_gather(0)
            wait_gather(0)
            wait_id(0)
            pltpu.sync_copy(rows[0], acc_sh.at[dv[0]], add=True)

        plsc.subcore_barrier()
        pltpu.sync_copy(acc_sh.at[pl.ds(s_ax * RPW, RPW)],
                        out_hbm.at[c_ax].at[pl.ds(s_ax * RPW, RPW)])

        @pl.when(s_ax == 0)
        def _():
            pltpu.sync_copy(acc_sh.at[pl.ds(NS * RPW, RTAIL)],
                            out_hbm.at[c_ax].at[pl.ds(NS * RPW, RTAIL)])

    return body(h, src, dst, zeros)


BLK = 1000


def _mlp_body(a_ref, w1_ref, b1_ref, w2_ref, b2_ref, sc_ref, sh_ref,
              out_ref):
    h = a_ref[0] + a_ref[1]
    t = jnp.dot(h, w1_ref[...], preferred_element_type=jnp.float32) + b1_ref[...]
    t = jnp.maximum(t, 0.0)
    t = jnp.dot(t, w2_ref[...], preferred_element_type=jnp.float32) + b2_ref[...]
    out_ref[...] = jnp.maximum(t * sc_ref[...] + sh_ref[...], 0.0)


def _mlp_tc(agg, w1, b1, w2, b2, scale, shift):
    grid = N // BLK
    full = lambda *_: (0, 0)
    return pl.pallas_call(
        _mlp_body,
        grid=(grid,),
        in_specs=[
            pl.BlockSpec((NC, BLK, D), lambda i: (0, i, 0)),
            pl.BlockSpec((D, D), full),
            pl.BlockSpec((1, D), full),
            pl.BlockSpec((D, D), full),
            pl.BlockSpec((1, D), full),
            pl.BlockSpec((1, D), full),
            pl.BlockSpec((1, D), full),
        ],
        out_specs=pl.BlockSpec((BLK, D), lambda i: (i, 0)),
        out_shape=jax.ShapeDtypeStruct((N, D), jnp.float32),
    )(agg, w1, b1, w2, b2, scale, shift)


def _mlp3_body(a_ref, w1_ref, b1_ref, w2_ref, b2_ref, sc_ref, sh_ref,
               wc_ref, bc_ref, out_ref, acc_ref):
    i = pl.program_id(0)
    h = a_ref[0] + a_ref[1]
    t = jnp.dot(h, w1_ref[...], preferred_element_type=jnp.float32) + b1_ref[...]
    t = jnp.maximum(t, 0.0)
    t = jnp.dot(t, w2_ref[...], preferred_element_type=jnp.float32) + b2_ref[...]
    t = jnp.maximum(t * sc_ref[...] + sh_ref[...], 0.0)
    psum = jnp.sum(t, axis=0, keepdims=True)

    @pl.when(i == 0)
    def _():
        acc_ref[...] = psum

    @pl.when(i > 0)
    def _():
        acc_ref[...] += psum

    @pl.when(i == pl.num_programs(0) - 1)
    def _():
        out_ref[...] = (jnp.dot(acc_ref[...], wc_ref[...],
                                preferred_element_type=jnp.float32)
                        + bc_ref[...])


def _mlp3_tc(agg, w1, b1, w2, b2, scale, shift, wc, bc):
    grid = N // BLK
    full = lambda *_: (0, 0)
    return pl.pallas_call(
        _mlp3_body,
        grid=(grid,),
        in_specs=[
            pl.BlockSpec((NC, BLK, D), lambda i: (0, i, 0)),
            pl.BlockSpec((D, D), full),
            pl.BlockSpec((1, D), full),
            pl.BlockSpec((D, D), full),
            pl.BlockSpec((1, D), full),
            pl.BlockSpec((1, D), full),
            pl.BlockSpec((1, D), full),
            pl.BlockSpec((D, D_OUT), full),
            pl.BlockSpec((1, D_OUT), full),
        ],
        out_specs=pl.BlockSpec((1, D_OUT), full),
        out_shape=jax.ShapeDtypeStruct((1, D_OUT), jnp.float32),
        scratch_shapes=[pltpu.VMEM((1, D), jnp.float32)],
    )(agg, w1, b1, w2, b2, scale, shift, wc, bc)


def kernel(x, edge_index,
           W1_0, b1_0, W2_0, b2_0, gamma_0, beta_0, mean_0, var_0,
           W1_1, b1_1, W2_1, b2_1, gamma_1, beta_1, mean_1, var_1,
           W1_2, b1_2, W2_2, b2_2, gamma_2, beta_2, mean_2, var_2,
           Wc, bc):
    src = edge_index[0]
    dst = edge_index[1]
    zeros = jnp.zeros((N, D), jnp.float32)

    def fold_bn(gamma, beta, mean, var):
        scale = gamma / jnp.sqrt(var + 1e-5)
        shift = beta - mean * scale
        return scale.reshape(1, D), shift.reshape(1, D)

    sc0, sh0 = fold_bn(gamma_0, beta_0, mean_0, var_0)
    sc1, sh1 = fold_bn(gamma_1, beta_1, mean_1, var_1)
    sc2, sh2 = fold_bn(gamma_2, beta_2, mean_2, var_2)

    h = x
    agg = _agg_sc(h, src, dst, zeros)
    h = _mlp_tc(agg, W1_0, b1_0.reshape(1, D), W2_0, b2_0.reshape(1, D), sc0, sh0)
    agg = _agg_sc(h, src, dst, zeros)
    h = _mlp_tc(agg, W1_1, b1_1.reshape(1, D), W2_1, b2_1.reshape(1, D), sc1, sh1)
    agg = _agg_sc(h, src, dst, zeros)
    return _mlp3_tc(agg, W1_2, b1_2.reshape(1, D), W2_2, b2_2.reshape(1, D),
                    sc2, sh2, Wc, bc.reshape(1, D_OUT))
